# Initial kernel scaffold; baseline (speedup 1.0000x reference)
import jax
import jax.numpy as jnp
from jax.experimental import pallas as pl


def kernel(h, edge_index, ppi_w, self_w, W, b):
    raise NotImplementedError("write your pallas kernel here")


# speedup vs baseline: 1.9909x; 1.9909x over previous
import functools

import jax
import jax.numpy as jnp
from jax import lax
from jax.experimental import pallas as pl
from jax.experimental.pallas import tpu as pltpu
from jax.experimental.pallas import tpu_sc as plsc

_NC = 2
_NS = 16
_C = 80
_HALF = 64


def _sc_layer(h2, gsrc, dst, sw, pw, zeros, n_nodes, n_edges):
    eps = n_edges // _NS
    nchunks = eps // _C
    npad = -(-n_nodes // (_NS * 8)) * (_NS * 8)
    rpt = npad // _NS

    mesh = plsc.VectorSubcoreMesh(core_axis_name="c", subcore_axis_name="s")

    @functools.partial(
        pl.kernel,
        mesh=mesh,
        compiler_params=pltpu.CompilerParams(
            needs_layout_passes=False, use_tc_tiling_on_sc=False),
        out_type=jax.ShapeDtypeStruct((2 * npad, 2 * _HALF), jnp.float32),
        scratch_types=[
            pltpu.VMEM((_C,), jnp.int32),
            pltpu.VMEM((_C,), jnp.int32),
            pltpu.VMEM((_C,), jnp.float32),
            pltpu.VMEM((_C,), jnp.float32),
            pltpu.VMEM((_C, _HALF), jnp.float32),
            pltpu.VMEM((_C, 2 * _HALF), jnp.float32),
            pltpu.VMEM_SHARED((npad, 2 * _HALF), jnp.float32),
            pltpu.SemaphoreType.DMA,
        ],
    )
    def sck(h2_hbm, gsrc_hbm, dst_hbm, sw_hbm, pw_hbm, z_hbm, out_hbm,
            gidx, dstv, swv, pwv, rows, upd, acc, sem):
        cid = lax.axis_index("c")
        sid = lax.axis_index("s")

        pltpu.sync_copy(z_hbm, acc.at[pl.ds(sid * rpt, rpt)])
        plsc.subcore_barrier()

        def chunk(k, carry):
            base = sid * eps + k * _C
            pltpu.sync_copy(gsrc_hbm.at[pl.ds(cid * n_edges + base, _C)], gidx)
            pltpu.sync_copy(dst_hbm.at[pl.ds(base, _C)], dstv)
            pltpu.sync_copy(sw_hbm.at[pl.ds(base, _C)], swv)
            pltpu.sync_copy(pw_hbm.at[pl.ds(base, _C)], pwv)
            pltpu.async_copy(h2_hbm.at[gidx], rows, sem).wait()

            def ebody(e, c2):
                splat = jnp.full((16,), e, jnp.int32)
                sws = plsc.load_gather(swv, [splat])
                pws = plsc.load_gather(pwv, [splat])
                for j in range(_HALF // 16):
                    r = rows[e, pl.ds(16 * j, 16)]
                    upd[e, pl.ds(16 * j, 16)] = r * sws
                    upd[e, pl.ds(_HALF + 16 * j, 16)] = r * pws
                return c2
            lax.fori_loop(0, _C, ebody, 0)

            pltpu.sync_copy(upd, acc.at[dstv], add=True)
            return carry
        lax.fori_loop(0, nchunks, chunk, 0)

        plsc.subcore_barrier()
        pltpu.sync_copy(acc.at[pl.ds(sid * rpt, rpt)],
                        out_hbm.at[pl.ds(cid * npad + sid * rpt, rpt)])

    return sck(h2, gsrc, dst, sw, pw, zeros)


def _tc_layer(acc3, w, b2, n_nodes, mid):
    nb = 1000
    grid = (n_nodes // nb,)

    def body(acc_ref, w_ref, b_ref, out_ref):
        a0 = acc_ref[0]
        a1 = acc_ref[1]
        res = jnp.concatenate([a0[:, :_HALF], a1[:, :_HALF]], axis=1)
        ppi = jnp.concatenate([a0[:, _HALF:], a1[:, _HALF:]], axis=1)
        hn = lax.dot_general(ppi, w_ref[...], (((1,), (1,)), ((), ())),
                             preferred_element_type=jnp.float32)
        hn = jnp.maximum(hn + b_ref[...], 0.0) + res
        if mid:
            out_ref[0] = hn[:, :_HALF]
            out_ref[1] = hn[:, _HALF:]
        else:
            out_ref[...] = hn

    in_specs = [
        pl.BlockSpec((2, nb, 2 * _HALF), lambda i: (0, i, 0)),
        pl.BlockSpec((2 * _HALF, 2 * _HALF), lambda i: (0, 0)),
        pl.BlockSpec((1, 2 * _HALF), lambda i: (0, 0)),
    ]
    if mid:
        out_spec = pl.BlockSpec((2, nb, _HALF), lambda i: (0, i, 0))
        out_shape = jax.ShapeDtypeStruct((2, n_nodes, _HALF), jnp.float32)
    else:
        out_spec = pl.BlockSpec((nb, 2 * _HALF), lambda i: (i, 0))
        out_shape = jax.ShapeDtypeStruct((n_nodes, 2 * _HALF), jnp.float32)
    return pl.pallas_call(body, grid=grid, in_specs=in_specs,
                          out_specs=out_spec, out_shape=out_shape)(acc3, w, b2)


def kernel(h, edge_index, ppi_w, self_w, W, b):
    n_nodes, d = h.shape
    n_layers = W.shape[0]
    n_edges = edge_index.shape[1]
    src = edge_index[0]
    dst = edge_index[1]

    h2 = jnp.concatenate([h[:, :_HALF], h[:, _HALF:]], axis=0)
    gsrc = jnp.concatenate([src, src + n_nodes])
    npad = -(-n_nodes // (_NS * 8)) * (_NS * 8)
    zeros = jnp.zeros((npad // _NS, d), jnp.float32)
    out = None
    for i in range(n_layers):
        accf = _sc_layer(h2, gsrc, dst, self_w[i], ppi_w[i], zeros,
                         n_nodes, n_edges)
        acc3 = accf.reshape(2, npad, d)
        b2 = b[i].reshape(1, d)
        if i + 1 < n_layers:
            h2 = _tc_layer(acc3, W[i], b2, n_nodes, True).reshape(2 * n_nodes, _HALF)
        else:
            out = _tc_layer(acc3, W[i], b2, n_nodes, False)
    return out


# speedup vs baseline: 4.2599x; 2.1397x over previous
import functools

import jax
import jax.numpy as jnp
from jax import lax
from jax.experimental import pallas as pl
from jax.experimental.pallas import tpu as pltpu
from jax.experimental.pallas import tpu_sc as plsc

_NC = 2
_NS = 16
_C = 80
_SB = 4000
_HALF = 64


def _sc_layer(h2, meta, wmeta, zeros, n_nodes, n_edges):
    eps = n_edges // _NS
    nsc = eps // _SB
    S = _SB // _C
    npad = -(-n_nodes // (_NS * 8)) * (_NS * 8)
    rpt = npad // _NS

    mesh = plsc.VectorSubcoreMesh(core_axis_name="c", subcore_axis_name="s")

    @functools.partial(
        pl.kernel,
        mesh=mesh,
        compiler_params=pltpu.CompilerParams(
            needs_layout_passes=False, use_tc_tiling_on_sc=False),
        out_type=jax.ShapeDtypeStruct((2 * npad, 2 * _HALF), jnp.float32),
        scratch_types=[
            pltpu.VMEM((2 * _SB,), jnp.int32),
            pltpu.VMEM((2 * _SB,), jnp.float32),
            pltpu.VMEM((_C, _HALF), jnp.float32),
            pltpu.VMEM((_C, _HALF), jnp.float32),
            pltpu.VMEM((_C, 2 * _HALF), jnp.float32),
            pltpu.VMEM((_C, 2 * _HALF), jnp.float32),
            pltpu.VMEM_SHARED((npad, 2 * _HALF), jnp.float32),
            pltpu.SemaphoreType.DMA,
            pltpu.SemaphoreType.DMA,
            pltpu.SemaphoreType.DMA,
            pltpu.SemaphoreType.DMA,
        ],
    )
    def sck(h2_hbm, meta_hbm, wmeta_hbm, z_hbm, out_hbm,
            mbuf, wbuf, rows0, rows1, upd0, upd1, acc,
            sg0, sg1, ss0, ss1):
        cid = lax.axis_index("c")
        sid = lax.axis_index("s")

        def gidx(k):
            return mbuf.at[pl.ds(k * _C, _C)]

        def didx(k):
            return mbuf.at[pl.ds(_SB + k * _C, _C)]

        def issue_gather(k, rref, sem):
            pltpu.async_copy(h2_hbm.at[gidx(k)], rref, sem)

        def wait_gather(rref, sem):
            pltpu.make_async_copy(h2_hbm.at[gidx(0)], rref, sem).wait()

        def issue_scatter(k, uref, sem):
            pltpu.async_copy(uref, acc.at[didx(k)], sem, add=True)

        def wait_scatter(uref, sem):
            pltpu.make_async_copy(uref, acc.at[didx(0)], sem).wait()

        def compute(k, rref, uref):
            so = k * _C
            po = _SB + k * _C

            def ebody(e, c2):
                sws = plsc.load_gather(wbuf, [jnp.full((16,), so + e, jnp.int32)])
                pws = plsc.load_gather(wbuf, [jnp.full((16,), po + e, jnp.int32)])
                for j in range(_HALF // 16):
                    r = rref[e, pl.ds(16 * j, 16)]
                    uref[e, pl.ds(16 * j, 16)] = r * sws
                    uref[e, pl.ds(_HALF + 16 * j, 16)] = r * pws
                return c2
            lax.fori_loop(0, _C, ebody, 0)

        pltpu.sync_copy(z_hbm, acc.at[pl.ds(sid * rpt, rpt)])
        plsc.subcore_barrier()

        def superchunk(t, carry):
            moff = ((cid * _NS + sid) * nsc + t) * (2 * _SB)
            woff = (sid * nsc + t) * (2 * _SB)
            pltpu.sync_copy(meta_hbm.at[pl.ds(moff, 2 * _SB)], mbuf)
            pltpu.sync_copy(wmeta_hbm.at[pl.ds(woff, 2 * _SB)], wbuf)

            issue_gather(0, rows0, sg0)
            issue_gather(1, rows1, sg1)
            wait_gather(rows0, sg0)
            compute(0, rows0, upd0)
            issue_scatter(0, upd0, ss0)
            issue_gather(2, rows0, sg0)
            wait_gather(rows1, sg1)
            compute(1, rows1, upd1)
            issue_scatter(1, upd1, ss1)

            def pair(m, c2):
                k0 = 2 + 2 * m
                issue_gather(k0 + 1, rows1, sg1)
                wait_scatter(upd0, ss0)
                wait_gather(rows0, sg0)
                compute(k0, rows0, upd0)
                issue_scatter(k0, upd0, ss0)

                @pl.when(k0 < S - 2)
                def _():
                    issue_gather(k0 + 2, rows0, sg0)
                wait_scatter(upd1, ss1)
                wait_gather(rows1, sg1)
                compute(k0 + 1, rows1, upd1)
                issue_scatter(k0 + 1, upd1, ss1)
                return c2
            lax.fori_loop(0, (S - 2) // 2, pair, 0)

            wait_scatter(upd0, ss0)
            wait_scatter(upd1, ss1)
            return carry
        lax.fori_loop(0, nsc, superchunk, 0)

        plsc.subcore_barrier()
        pltpu.sync_copy(acc.at[pl.ds(sid * rpt, rpt)],
                        out_hbm.at[pl.ds(cid * npad + sid * rpt, rpt)])

    return sck(h2, meta, wmeta, zeros)


def _tc_layer(acc3, w, b2, n_nodes, mid):
    nb = 1000
    grid = (n_nodes // nb,)

    def body(acc_ref, w_ref, b_ref, out_ref):
        a0 = acc_ref[0]
        a1 = acc_ref[1]
        res = jnp.concatenate([a0[:, :_HALF], a1[:, :_HALF]], axis=1)
        ppi = jnp.concatenate([a0[:, _HALF:], a1[:, _HALF:]], axis=1)
        hn = lax.dot_general(ppi, w_ref[...], (((1,), (1,)), ((), ())),
                             preferred_element_type=jnp.float32)
        hn = jnp.maximum(hn + b_ref[...], 0.0) + res
        if mid:
            out_ref[0] = hn[:, :_HALF]
            out_ref[1] = hn[:, _HALF:]
        else:
            out_ref[...] = hn

    in_specs = [
        pl.BlockSpec((2, nb, 2 * _HALF), lambda i: (0, i, 0)),
        pl.BlockSpec((2 * _HALF, 2 * _HALF), lambda i: (0, 0)),
        pl.BlockSpec((1, 2 * _HALF), lambda i: (0, 0)),
    ]
    if mid:
        out_spec = pl.BlockSpec((2, nb, _HALF), lambda i: (0, i, 0))
        out_shape = jax.ShapeDtypeStruct((2, n_nodes, _HALF), jnp.float32)
    else:
        out_spec = pl.BlockSpec((nb, 2 * _HALF), lambda i: (i, 0))
        out_shape = jax.ShapeDtypeStruct((n_nodes, 2 * _HALF), jnp.float32)
    return pl.pallas_call(body, grid=grid, in_specs=in_specs,
                          out_specs=out_spec, out_shape=out_shape)(acc3, w, b2)


def kernel(h, edge_index, ppi_w, self_w, W, b):
    n_nodes, d = h.shape
    n_layers = W.shape[0]
    n_edges = edge_index.shape[1]
    src = edge_index[0]
    dst = edge_index[1]

    eps = n_edges // _NS
    nsc = eps // _SB
    h2 = jnp.concatenate([h[:, :_HALF], h[:, _HALF:]], axis=0)
    gs = src.reshape(_NS, nsc, _SB)
    ds_ = dst.reshape(_NS, nsc, _SB)
    meta = jnp.stack([jnp.stack([gs, ds_], axis=2),
                      jnp.stack([gs + n_nodes, ds_], axis=2)]).reshape(-1)
    npad = -(-n_nodes // (_NS * 8)) * (_NS * 8)
    zeros = jnp.zeros((npad // _NS, d), jnp.float32)
    out = None
    for i in range(n_layers):
        wmeta = jnp.stack([self_w[i].reshape(_NS, nsc, _SB),
                           ppi_w[i].reshape(_NS, nsc, _SB)], axis=2).reshape(-1)
        accf = _sc_layer(h2, meta, wmeta, zeros, n_nodes, n_edges)
        acc3 = accf.reshape(2, npad, d)
        b2 = b[i].reshape(1, d)
        if i + 1 < n_layers:
            h2 = _tc_layer(acc3, W[i], b2, n_nodes, True).reshape(2 * n_nodes, _HALF)
        else:
            out = _tc_layer(acc3, W[i], b2, n_nodes, False)
    return out
